# baseline (speedup 1.0000x reference)
import functools

import jax
import jax.numpy as jnp
from jax import lax
from jax.experimental import pallas as pl
from jax.experimental.pallas import tpu as pltpu
from jax.experimental.pallas import tpu_sc as plsc

_T = 2048
_D = 1024
_F = 1024
_E = 8
_M = 128
_CAP = 4096 + _E * _M
_NBLK = _CAP // _M


def _router_body(x_ref, wg_ref, xa_ref, xb_ref, pos_ref, bexp_ref):
    x = x_ref[...]
    wg = wg_ref[...]
    logits = lax.dot_general(
        x, wg, (((1,), (0,)), ((), ())),
        preferred_element_type=jnp.float32,
        precision=lax.Precision.HIGHEST,
    )
    m = jnp.max(logits, axis=1, keepdims=True)
    p = jnp.exp(logits - m)
    p = p / jnp.sum(p, axis=1, keepdims=True)
    lane = lax.broadcasted_iota(jnp.int32, p.shape, 1)
    m1 = jnp.max(p, axis=1, keepdims=True)
    i1 = jnp.min(jnp.where(p == m1, lane, _E), axis=1, keepdims=True)
    p2 = jnp.where(lane == i1, -1.0, p)
    m2 = jnp.max(p2, axis=1, keepdims=True)
    i2 = jnp.min(jnp.where(p2 == m2, lane, _E), axis=1, keepdims=True)
    den = m1 + m2 + 1e-9
    g1 = m1 / den
    g2 = m2 / den
    xa_ref[...] = g1 * x
    xb_ref[...] = g2 * x

    onehot = (jnp.where(lane == i1, 1.0, 0.0)
              + jnp.where(lane == i2, 1.0, 0.0)).astype(jnp.bfloat16)
    ri = lax.broadcasted_iota(jnp.int32, (256, 256), 0)
    ci = lax.broadcasted_iota(jnp.int32, (256, 256), 1)
    ltri = (ci < ri).astype(jnp.bfloat16)
    parts = []
    carry = jnp.zeros((1, _E), jnp.float32)
    for j in range(_T // 256):
        oh = onehot[j * 256:(j + 1) * 256]
        r = lax.dot_general(
            ltri, oh, (((1,), (0,)), ((), ())),
            preferred_element_type=jnp.float32,
        ) + carry
        parts.append(r)
        carry = carry + jnp.sum(oh.astype(jnp.float32), axis=0, keepdims=True)
    ranks = jnp.concatenate(parts, axis=0)
    counts = carry
    pt = jnp.ceil(counts / _M) * _M
    r8 = lax.broadcasted_iota(jnp.int32, (_E, _E), 0)
    c8 = lax.broadcasted_iota(jnp.int32, (_E, _E), 1)
    utri = (r8 < c8).astype(jnp.float32)
    off = lax.dot_general(
        pt, utri, (((1,), (0,)), ((), ())),
        preferred_element_type=jnp.float32,
        precision=lax.Precision.HIGHEST,
    )
    off_b = jnp.broadcast_to(off, (_T, _E))
    rank1 = jnp.sum(jnp.where(lane == i1, ranks, 0.0), axis=1, keepdims=True)
    rank2 = jnp.sum(jnp.where(lane == i2, ranks, 0.0), axis=1, keepdims=True)
    off1 = jnp.sum(jnp.where(lane == i1, off_b, 0.0), axis=1, keepdims=True)
    off2 = jnp.sum(jnp.where(lane == i2, off_b, 0.0), axis=1, keepdims=True)
    pos1 = (off1 + rank1).astype(jnp.int32)
    pos2 = (off2 + rank2).astype(jnp.int32)
    pos_ref[...] = jnp.concatenate([pos1, pos2], axis=1)

    brow = lax.broadcasted_iota(jnp.int32, (_NBLK, _E), 0).astype(jnp.float32) * _M
    blane = lax.broadcasted_iota(jnp.int32, (_NBLK, _E), 1)
    off_nb = jnp.broadcast_to(off, (_NBLK, _E))
    pt_nb = jnp.broadcast_to(pt, (_NBLK, _E))
    member = jnp.logical_and(brow >= off_nb, brow < off_nb + pt_nb)
    bexp_ref[...] = jnp.sum(jnp.where(member, blane, 0), axis=1, keepdims=True)


def _router(xt, wg):
    return pl.pallas_call(
        _router_body,
        out_shape=(
            jax.ShapeDtypeStruct((_T, _D), jnp.float32),
            jax.ShapeDtypeStruct((_T, _D), jnp.float32),
            jax.ShapeDtypeStruct((_T, 2), jnp.int32),
            jax.ShapeDtypeStruct((_NBLK, 1), jnp.int32),
        ),
    )(xt, wg)


def _gmm_body(bexp_ref, xs_ref, w1_ref, w2_ref, y_ref):
    xb = xs_ref[...].astype(jnp.bfloat16)
    w1 = w1_ref[0].astype(jnp.bfloat16)
    w2 = w2_ref[0].astype(jnp.bfloat16)
    h = lax.dot_general(
        xb, w1, (((1,), (0,)), ((), ())),
        preferred_element_type=jnp.float32,
    )
    h = jnp.maximum(h, 0.0).astype(jnp.bfloat16)
    y_ref[...] = lax.dot_general(
        h, w2, (((1,), (0,)), ((), ())),
        preferred_element_type=jnp.float32,
    )


def _gmm(bexp, xs, w1, w2):
    grid_spec = pltpu.PrefetchScalarGridSpec(
        num_scalar_prefetch=1,
        grid=(_NBLK,),
        in_specs=[
            pl.BlockSpec((_M, _D), lambda b, be: (b, 0)),
            pl.BlockSpec((1, _D, _F), lambda b, be: (be[b], 0, 0)),
            pl.BlockSpec((1, _F, _D), lambda b, be: (be[b], 0, 0)),
        ],
        out_specs=pl.BlockSpec((_M, _D), lambda b, be: (b, 0)),
    )
    return pl.pallas_call(
        _gmm_body,
        grid_spec=grid_spec,
        out_shape=jax.ShapeDtypeStruct((_CAP, _D), jnp.float32),
    )(bexp, xs, w1, w2)


_NC, _NS = 2, 16
_NW = _NC * _NS
_TPW = _T // _NW


def _sc_scatter(xa, xb, pos_e, pos_o):
    mesh = plsc.VectorSubcoreMesh(core_axis_name="c", subcore_axis_name="s")

    @functools.partial(
        pl.kernel, mesh=mesh,
        out_type=jax.ShapeDtypeStruct((_CAP, _D), jnp.float32),
        scratch_types=[
            pltpu.VMEM((_TPW,), jnp.int32),
            pltpu.VMEM((_TPW,), jnp.int32),
            pltpu.VMEM((_TPW, _D), jnp.float32),
            pltpu.SemaphoreType.DMA,
        ],
    )
    def k(xa_hbm, xb_hbm, pe_hbm, po_hbm, out_hbm, pev, pov, xv, sem):
        wid = lax.axis_index("s") * _NC + lax.axis_index("c")
        base = wid * _TPW
        pltpu.sync_copy(pe_hbm.at[pl.ds(base, _TPW)], pev)
        pltpu.sync_copy(po_hbm.at[pl.ds(base, _TPW)], pov)
        pltpu.sync_copy(xa_hbm.at[pl.ds(base, _TPW)], xv)
        pltpu.async_copy(xv, out_hbm.at[pev], sem).wait()
        pltpu.sync_copy(xb_hbm.at[pl.ds(base, _TPW)], xv)
        pltpu.async_copy(xv, out_hbm.at[pov], sem).wait()

    return k(xa, xb, pos_e, pos_o)


def _sc_combine(y, pos_e, pos_o):
    mesh = plsc.VectorSubcoreMesh(core_axis_name="c", subcore_axis_name="s")
    half = _TPW // 2

    @functools.partial(
        pl.kernel, mesh=mesh,
        out_type=jax.ShapeDtypeStruct((_T, _D), jnp.float32),
        scratch_types=[
            pltpu.VMEM((_TPW,), jnp.int32),
            pltpu.VMEM((_TPW,), jnp.int32),
            pltpu.VMEM((half, _D), jnp.float32),
            pltpu.VMEM((half, _D), jnp.float32),
            pltpu.SemaphoreType.DMA,
            pltpu.SemaphoreType.DMA,
        ],
    )
    def k(y_hbm, pe_hbm, po_hbm, out_hbm, pev, pov, rowsa, rowsb, sem1, sem2):
        wid = lax.axis_index("s") * _NC + lax.axis_index("c")
        base = wid * _TPW
        pltpu.sync_copy(pe_hbm.at[pl.ds(base, _TPW)], pev)
        pltpu.sync_copy(po_hbm.at[pl.ds(base, _TPW)], pov)
        for c in range(_TPW // half):
            ca = pltpu.async_copy(
                y_hbm.at[pev.at[pl.ds(half * c, half)]], rowsa, sem1)
            cb = pltpu.async_copy(
                y_hbm.at[pov.at[pl.ds(half * c, half)]], rowsb, sem2)
            ca.wait()
            cb.wait()

            def body(j, _):
                for l in range(_D // 16):
                    a = rowsa[j, pl.ds(16 * l, 16)]
                    b = rowsb[j, pl.ds(16 * l, 16)]
                    rowsa[j, pl.ds(16 * l, 16)] = a + b
                return 0

            lax.fori_loop(0, half, body, 0)
            pltpu.sync_copy(rowsa, out_hbm.at[pl.ds(base + half * c, half)])

    return k(y, pos_e, pos_o)


def _loss_body(y_ref, x_ref, m_ref, out_ref, acc_ref):
    t = pl.program_id(0)
    blk = x_ref[...] + m_ref[...]
    s = jnp.sum(blk, axis=0, keepdims=True)

    @pl.when(t == 0)
    def _():
        acc_ref[...] = s

    @pl.when(t > 0)
    def _():
        acc_ref[...] = acc_ref[...] + s

    @pl.when(t == pl.num_programs(0) - 1)
    def _():
        sent = acc_ref[...] / float(_T)
        mx = jnp.max(sent)
        lse = jnp.log(jnp.sum(jnp.exp(sent - mx))) + mx
        yv = y_ref[0]
        lane = lax.broadcasted_iota(jnp.int32, sent.shape, 1)
        picked = jnp.sum(jnp.where(lane == yv, sent, 0.0))
        out_ref[0, 0] = lse - picked


def _loss(y, xt, moe_out, bt=512):
    nt = _T // bt
    return pl.pallas_call(
        _loss_body,
        grid=(nt,),
        in_specs=[
            pl.BlockSpec(memory_space=pltpu.SMEM),
            pl.BlockSpec((bt, _D), lambda t: (t, 0)),
            pl.BlockSpec((bt, _D), lambda t: (t, 0)),
        ],
        out_specs=pl.BlockSpec(memory_space=pltpu.SMEM),
        out_shape=jax.ShapeDtypeStruct((1, 1), jnp.float32),
        scratch_shapes=[pltpu.VMEM((1, _D), jnp.float32)],
    )(y, xt, moe_out)


def _moe_layer(xt, wg, w1, w2):
    xa, xb, pos, bexp = _router(xt, wg)
    pos_e, pos_o = pos[:, 0], pos[:, 1]
    xs = _sc_scatter(xa, xb, pos_e, pos_o)
    ys = _gmm(bexp.reshape(-1), xs, w1, w2)
    return _sc_combine(ys, pos_e, pos_o)


def kernel(x, y, Wg1, W1a, W1b, Wg2, W2a, W2b):
    xt = x.reshape(_T, _D)
    m1 = _moe_layer(xt, Wg1, W1a, W1b)
    m2 = _moe_layer(m1, Wg2, W2a, W2b)
    out = _loss(y.astype(jnp.int32), xt, m2)
    return out[0, 0]


# speedup vs baseline: 1.0181x; 1.0181x over previous
import functools

import jax
import jax.numpy as jnp
from jax import lax
from jax.experimental import pallas as pl
from jax.experimental.pallas import tpu as pltpu
from jax.experimental.pallas import tpu_sc as plsc

_T = 2048
_D = 1024
_F = 1024
_E = 8
_M = 128
_CAP = 4096 + _E * _M
_NBLK = _CAP // _M


def _router_body(x_ref, wg_ref, ge_ref, go_ref, pos_ref, bexp_ref, xsum_ref):
    x = x_ref[...]
    wg = wg_ref[...]
    logits = lax.dot_general(
        x, wg, (((1,), (0,)), ((), ())),
        preferred_element_type=jnp.float32,
        precision=lax.Precision.HIGHEST,
    )
    m = jnp.max(logits, axis=1, keepdims=True)
    p = jnp.exp(logits - m)
    p = p / jnp.sum(p, axis=1, keepdims=True)
    lane = lax.broadcasted_iota(jnp.int32, p.shape, 1)
    m1 = jnp.max(p, axis=1, keepdims=True)
    i1 = jnp.min(jnp.where(p == m1, lane, _E), axis=1, keepdims=True)
    p2 = jnp.where(lane == i1, -1.0, p)
    m2 = jnp.max(p2, axis=1, keepdims=True)
    i2 = jnp.min(jnp.where(p2 == m2, lane, _E), axis=1, keepdims=True)
    den = m1 + m2 + 1e-9
    g1 = m1 / den
    g2 = m2 / den
    ge_ref[...] = jnp.broadcast_to(g1, (_T, 128))
    go_ref[...] = jnp.broadcast_to(g2, (_T, 128))
    xsum_ref[...] = jnp.sum(x, axis=0, keepdims=True)

    onehot = (jnp.where(lane == i1, 1.0, 0.0)
              + jnp.where(lane == i2, 1.0, 0.0)).astype(jnp.bfloat16)
    ri = lax.broadcasted_iota(jnp.int32, (256, 256), 0)
    ci = lax.broadcasted_iota(jnp.int32, (256, 256), 1)
    ltri = (ci < ri).astype(jnp.bfloat16)
    parts = []
    carry = jnp.zeros((1, _E), jnp.float32)
    for j in range(_T // 256):
        oh = onehot[j * 256:(j + 1) * 256]
        r = lax.dot_general(
            ltri, oh, (((1,), (0,)), ((), ())),
            preferred_element_type=jnp.float32,
        ) + carry
        parts.append(r)
        carry = carry + jnp.sum(oh.astype(jnp.float32), axis=0, keepdims=True)
    ranks = jnp.concatenate(parts, axis=0)
    counts = carry
    pt = jnp.ceil(counts / _M) * _M
    r8 = lax.broadcasted_iota(jnp.int32, (_E, _E), 0)
    c8 = lax.broadcasted_iota(jnp.int32, (_E, _E), 1)
    utri = (r8 < c8).astype(jnp.float32)
    off = lax.dot_general(
        pt, utri, (((1,), (0,)), ((), ())),
        preferred_element_type=jnp.float32,
        precision=lax.Precision.HIGHEST,
    )
    off_b = jnp.broadcast_to(off, (_T, _E))
    rank1 = jnp.sum(jnp.where(lane == i1, ranks, 0.0), axis=1, keepdims=True)
    rank2 = jnp.sum(jnp.where(lane == i2, ranks, 0.0), axis=1, keepdims=True)
    off1 = jnp.sum(jnp.where(lane == i1, off_b, 0.0), axis=1, keepdims=True)
    off2 = jnp.sum(jnp.where(lane == i2, off_b, 0.0), axis=1, keepdims=True)
    pos1 = (off1 + rank1).astype(jnp.int32)
    pos2 = (off2 + rank2).astype(jnp.int32)
    pos_ref[...] = jnp.concatenate([pos1, pos2], axis=1)

    brow = lax.broadcasted_iota(jnp.int32, (_NBLK, _E), 0).astype(jnp.float32) * _M
    blane = lax.broadcasted_iota(jnp.int32, (_NBLK, _E), 1)
    off_nb = jnp.broadcast_to(off, (_NBLK, _E))
    pt_nb = jnp.broadcast_to(pt, (_NBLK, _E))
    member = jnp.logical_and(brow >= off_nb, brow < off_nb + pt_nb)
    bexp_ref[...] = jnp.sum(jnp.where(member, blane, 0), axis=1, keepdims=True)


def _router(xt, wg):
    return pl.pallas_call(
        _router_body,
        out_shape=(
            jax.ShapeDtypeStruct((_T, 128), jnp.float32),
            jax.ShapeDtypeStruct((_T, 128), jnp.float32),
            jax.ShapeDtypeStruct((_T, 2), jnp.int32),
            jax.ShapeDtypeStruct((_NBLK, 1), jnp.int32),
            jax.ShapeDtypeStruct((1, _D), jnp.float32),
        ),
    )(xt, wg)


def _gmm_body(bexp_ref, xs_ref, gs_ref, w1_ref, w2_ref, y_ref):
    xb = xs_ref[...].astype(jnp.bfloat16)
    w1 = w1_ref[0].astype(jnp.bfloat16)
    w2 = w2_ref[0].astype(jnp.bfloat16)
    h = lax.dot_general(
        xb, w1, (((1,), (0,)), ((), ())),
        preferred_element_type=jnp.float32,
    )
    g = gs_ref[...][:, 0:1]
    h = (jnp.maximum(h, 0.0) * g).astype(jnp.bfloat16)
    y_ref[...] = lax.dot_general(
        h, w2, (((1,), (0,)), ((), ())),
        preferred_element_type=jnp.float32,
    )


def _gmm(bexp, xs, gs, w1, w2):
    grid_spec = pltpu.PrefetchScalarGridSpec(
        num_scalar_prefetch=1,
        grid=(_NBLK,),
        in_specs=[
            pl.BlockSpec((_M, _D), lambda b, be: (b, 0)),
            pl.BlockSpec((_M, 128), lambda b, be: (b, 0)),
            pl.BlockSpec((1, _D, _F), lambda b, be: (be[b], 0, 0)),
            pl.BlockSpec((1, _F, _D), lambda b, be: (be[b], 0, 0)),
        ],
        out_specs=pl.BlockSpec((_M, _D), lambda b, be: (b, 0)),
    )
    return pl.pallas_call(
        _gmm_body,
        grid_spec=grid_spec,
        out_shape=jax.ShapeDtypeStruct((_CAP, _D), jnp.float32),
    )(bexp, xs, gs, w1, w2)


_NC, _NS = 2, 16
_NW = _NC * _NS
_TPW = _T // _NW


def _sc_scatter(x2d, ge, go, pos_e, pos_o):
    mesh = plsc.VectorSubcoreMesh(core_axis_name="c", subcore_axis_name="s")

    @functools.partial(
        pl.kernel, mesh=mesh,
        out_type=(
            jax.ShapeDtypeStruct((_CAP, _D), jnp.float32),
            jax.ShapeDtypeStruct((_CAP, 128), jnp.float32),
        ),
        scratch_types=[
            pltpu.VMEM((_TPW,), jnp.int32),
            pltpu.VMEM((_TPW,), jnp.int32),
            pltpu.VMEM((_TPW, _D), jnp.float32),
            pltpu.VMEM((_TPW, 128), jnp.float32),
            pltpu.VMEM((_TPW, 128), jnp.float32),
            pltpu.SemaphoreType.DMA,
            pltpu.SemaphoreType.DMA,
            pltpu.SemaphoreType.DMA,
            pltpu.SemaphoreType.DMA,
        ],
    )
    def k(x_hbm, ge_hbm, go_hbm, pe_hbm, po_hbm, xs_hbm, gs_hbm,
          pev, pov, xv, gev, gov, s1, s2, s3, s4):
        wid = lax.axis_index("s") * _NC + lax.axis_index("c")
        base = wid * _TPW
        pltpu.sync_copy(pe_hbm.at[pl.ds(base, _TPW)], pev)
        pltpu.sync_copy(po_hbm.at[pl.ds(base, _TPW)], pov)
        pltpu.sync_copy(x_hbm.at[pl.ds(base, _TPW)], xv)
        pltpu.sync_copy(ge_hbm.at[pl.ds(base, _TPW)], gev)
        pltpu.sync_copy(go_hbm.at[pl.ds(base, _TPW)], gov)
        c1 = pltpu.async_copy(xv, xs_hbm.at[pev], s1)
        c2 = pltpu.async_copy(xv, xs_hbm.at[pov], s2)
        c3 = pltpu.async_copy(gev, gs_hbm.at[pev], s3)
        c4 = pltpu.async_copy(gov, gs_hbm.at[pov], s4)
        c1.wait()
        c2.wait()
        c3.wait()
        c4.wait()

    return k(x2d, ge, go, pos_e, pos_o)


def _sc_combine(y, pos_e, pos_o):
    mesh = plsc.VectorSubcoreMesh(core_axis_name="c", subcore_axis_name="s")
    half = _TPW // 2

    @functools.partial(
        pl.kernel, mesh=mesh,
        out_type=jax.ShapeDtypeStruct((_T, _D), jnp.float32),
        scratch_types=[
            pltpu.VMEM((_TPW,), jnp.int32),
            pltpu.VMEM((_TPW,), jnp.int32),
            pltpu.VMEM((half, _D), jnp.float32),
            pltpu.VMEM((half, _D), jnp.float32),
            pltpu.SemaphoreType.DMA,
            pltpu.SemaphoreType.DMA,
        ],
    )
    def k(y_hbm, pe_hbm, po_hbm, out_hbm, pev, pov, rowsa, rowsb, sem1, sem2):
        wid = lax.axis_index("s") * _NC + lax.axis_index("c")
        base = wid * _TPW
        pltpu.sync_copy(pe_hbm.at[pl.ds(base, _TPW)], pev)
        pltpu.sync_copy(po_hbm.at[pl.ds(base, _TPW)], pov)
        for c in range(_TPW // half):
            ca = pltpu.async_copy(
                y_hbm.at[pev.at[pl.ds(half * c, half)]], rowsa, sem1)
            cb = pltpu.async_copy(
                y_hbm.at[pov.at[pl.ds(half * c, half)]], rowsb, sem2)
            ca.wait()
            cb.wait()

            def body(j, _):
                for l in range(_D // 16):
                    a = rowsa[j, pl.ds(16 * l, 16)]
                    b = rowsb[j, pl.ds(16 * l, 16)]
                    rowsa[j, pl.ds(16 * l, 16)] = a + b
                return 0

            lax.fori_loop(0, half, body, 0)
            pltpu.sync_copy(rowsa, out_hbm.at[pl.ds(base + half * c, half)])

    return k(y, pos_e, pos_o)


def _sc_combine_reduce(y, pos_e, pos_o):
    mesh = plsc.VectorSubcoreMesh(core_axis_name="c", subcore_axis_name="s")
    half = _TPW // 2

    @functools.partial(
        pl.kernel, mesh=mesh,
        out_type=jax.ShapeDtypeStruct((_NW, _D), jnp.float32),
        scratch_types=[
            pltpu.VMEM((_TPW,), jnp.int32),
            pltpu.VMEM((_TPW,), jnp.int32),
            pltpu.VMEM((half, _D), jnp.float32),
            pltpu.VMEM((half, _D), jnp.float32),
            pltpu.VMEM((1, _D), jnp.float32),
            pltpu.SemaphoreType.DMA,
            pltpu.SemaphoreType.DMA,
        ],
    )
    def k(y_hbm, pe_hbm, po_hbm, out_hbm, pev, pov, rowsa, rowsb, acc,
          sem1, sem2):
        wid = lax.axis_index("s") * _NC + lax.axis_index("c")
        base = wid * _TPW
        pltpu.sync_copy(pe_hbm.at[pl.ds(base, _TPW)], pev)
        pltpu.sync_copy(po_hbm.at[pl.ds(base, _TPW)], pov)
        zero = jnp.zeros((16,), jnp.float32)
        for l in range(_D // 16):
            acc[0, pl.ds(16 * l, 16)] = zero
        for c in range(_TPW // half):
            ca = pltpu.async_copy(
                y_hbm.at[pev.at[pl.ds(half * c, half)]], rowsa, sem1)
            cb = pltpu.async_copy(
                y_hbm.at[pov.at[pl.ds(half * c, half)]], rowsb, sem2)
            ca.wait()
            cb.wait()

            def body(j, _):
                for l in range(_D // 16):
                    a = rowsa[j, pl.ds(16 * l, 16)]
                    b = rowsb[j, pl.ds(16 * l, 16)]
                    acc[0, pl.ds(16 * l, 16)] = acc[0, pl.ds(16 * l, 16)] + a + b
                return 0

            lax.fori_loop(0, half, body, 0)
        pltpu.sync_copy(acc, out_hbm.at[pl.ds(wid, 1)])

    return k(y, pos_e, pos_o)


def _loss_body(y_ref, xsum_ref, part_ref, out_ref):
    sent = (xsum_ref[...] + jnp.sum(part_ref[...], axis=0, keepdims=True))
    sent = sent / float(_T)
    mx = jnp.max(sent)
    lse = jnp.log(jnp.sum(jnp.exp(sent - mx))) + mx
    yv = y_ref[0]
    lane = lax.broadcasted_iota(jnp.int32, sent.shape, 1)
    picked = jnp.sum(jnp.where(lane == yv, sent, 0.0))
    out_ref[0, 0] = lse - picked


def _loss(y, xsum, partials):
    return pl.pallas_call(
        _loss_body,
        in_specs=[
            pl.BlockSpec(memory_space=pltpu.SMEM),
            pl.BlockSpec((1, _D), lambda: (0, 0)),
            pl.BlockSpec((_NW, _D), lambda: (0, 0)),
        ],
        out_specs=pl.BlockSpec(memory_space=pltpu.SMEM),
        out_shape=jax.ShapeDtypeStruct((1, 1), jnp.float32),
    )(y, xsum, partials)


def kernel(x, y, Wg1, W1a, W1b, Wg2, W2a, W2b):
    xt = x.reshape(_T, _D)
    ge1, go1, pos1, bexp1, xsum = _router(xt, Wg1)
    pe1, po1 = pos1[:, 0], pos1[:, 1]
    xs1, gs1 = _sc_scatter(xt, ge1, go1, pe1, po1)
    y1 = _gmm(bexp1.reshape(-1), xs1, gs1, W1a, W1b)
    m1 = _sc_combine(y1, pe1, po1)
    ge2, go2, pos2, bexp2, _ = _router(m1, Wg2)
    pe2, po2 = pos2[:, 0], pos2[:, 1]
    xs2, gs2 = _sc_scatter(m1, ge2, go2, pe2, po2)
    y2 = _gmm(bexp2.reshape(-1), xs2, gs2, W2a, W2b)
    part = _sc_combine_reduce(y2, pe2, po2)
    out = _loss(y.astype(jnp.int32), xsum, part)
    return out[0, 0]


# speedup vs baseline: 1.0610x; 1.0421x over previous
import functools

import jax
import jax.numpy as jnp
from jax import lax
from jax.experimental import pallas as pl
from jax.experimental.pallas import tpu as pltpu
from jax.experimental.pallas import tpu_sc as plsc

_T = 2048
_D = 1024
_F = 1024
_E = 8
_M = 256
_CAP = 4096 + _E * _M
_NBLK = _CAP // _M


def _router_body(x_ref, wg_ref, ge_ref, go_ref, pos_ref, bexp_ref, xsum_ref):
    x = x_ref[...]
    wg = wg_ref[...]
    logits = lax.dot_general(
        x, wg, (((1,), (0,)), ((), ())),
        preferred_element_type=jnp.float32,
        precision=lax.Precision.HIGHEST,
    )
    m = jnp.max(logits, axis=1, keepdims=True)
    p = jnp.exp(logits - m)
    p = p / jnp.sum(p, axis=1, keepdims=True)
    lane = lax.broadcasted_iota(jnp.int32, p.shape, 1)
    m1 = jnp.max(p, axis=1, keepdims=True)
    i1 = jnp.min(jnp.where(p == m1, lane, _E), axis=1, keepdims=True)
    p2 = jnp.where(lane == i1, -1.0, p)
    m2 = jnp.max(p2, axis=1, keepdims=True)
    i2 = jnp.min(jnp.where(p2 == m2, lane, _E), axis=1, keepdims=True)
    den = m1 + m2 + 1e-9
    g1 = m1 / den
    g2 = m2 / den
    ge_ref[...] = jnp.broadcast_to(g1, (_T, 128))
    go_ref[...] = jnp.broadcast_to(g2, (_T, 128))
    xsum_ref[...] = jnp.sum(x, axis=0, keepdims=True)

    onehot = (jnp.where(lane == i1, 1.0, 0.0)
              + jnp.where(lane == i2, 1.0, 0.0)).astype(jnp.bfloat16)
    ri = lax.broadcasted_iota(jnp.int32, (256, 256), 0)
    ci = lax.broadcasted_iota(jnp.int32, (256, 256), 1)
    ltri = (ci < ri).astype(jnp.bfloat16)
    parts = []
    carry = jnp.zeros((1, _E), jnp.float32)
    for j in range(_T // 256):
        oh = onehot[j * 256:(j + 1) * 256]
        r = lax.dot_general(
            ltri, oh, (((1,), (0,)), ((), ())),
            preferred_element_type=jnp.float32,
        ) + carry
        parts.append(r)
        carry = carry + jnp.sum(oh.astype(jnp.float32), axis=0, keepdims=True)
    ranks = jnp.concatenate(parts, axis=0)
    counts = carry
    pt = jnp.ceil(counts / _M) * _M
    r8 = lax.broadcasted_iota(jnp.int32, (_E, _E), 0)
    c8 = lax.broadcasted_iota(jnp.int32, (_E, _E), 1)
    utri = (r8 < c8).astype(jnp.float32)
    off = lax.dot_general(
        pt, utri, (((1,), (0,)), ((), ())),
        preferred_element_type=jnp.float32,
        precision=lax.Precision.HIGHEST,
    )
    off_b = jnp.broadcast_to(off, (_T, _E))
    rank1 = jnp.sum(jnp.where(lane == i1, ranks, 0.0), axis=1, keepdims=True)
    rank2 = jnp.sum(jnp.where(lane == i2, ranks, 0.0), axis=1, keepdims=True)
    off1 = jnp.sum(jnp.where(lane == i1, off_b, 0.0), axis=1, keepdims=True)
    off2 = jnp.sum(jnp.where(lane == i2, off_b, 0.0), axis=1, keepdims=True)
    pos1 = (off1 + rank1).astype(jnp.int32)
    pos2 = (off2 + rank2).astype(jnp.int32)
    pos_ref[...] = jnp.concatenate([pos1, pos2], axis=1)

    brow = lax.broadcasted_iota(jnp.int32, (_NBLK, _E), 0).astype(jnp.float32) * _M
    blane = lax.broadcasted_iota(jnp.int32, (_NBLK, _E), 1)
    off_nb = jnp.broadcast_to(off, (_NBLK, _E))
    pt_nb = jnp.broadcast_to(pt, (_NBLK, _E))
    member = jnp.logical_and(brow >= off_nb, brow < off_nb + pt_nb)
    bexp_ref[...] = jnp.sum(jnp.where(member, blane, 0), axis=1, keepdims=True)


def _router(xt, wg):
    return pl.pallas_call(
        _router_body,
        out_shape=(
            jax.ShapeDtypeStruct((_T, 128), jnp.float32),
            jax.ShapeDtypeStruct((_T, 128), jnp.float32),
            jax.ShapeDtypeStruct((_T, 2), jnp.int32),
            jax.ShapeDtypeStruct((_NBLK, 1), jnp.int32),
            jax.ShapeDtypeStruct((1, _D), jnp.float32),
        ),
    )(xt, wg)


def _gmm_body(bexp_ref, xs_ref, gs_ref, w1_ref, w2_ref, y_ref):
    xb = xs_ref[...].astype(jnp.bfloat16)
    w1 = w1_ref[0].astype(jnp.bfloat16)
    w2 = w2_ref[0].astype(jnp.bfloat16)
    h = lax.dot_general(
        xb, w1, (((1,), (0,)), ((), ())),
        preferred_element_type=jnp.float32,
    )
    g = gs_ref[...][:, 0:1]
    h = (jnp.maximum(h, 0.0) * g).astype(jnp.bfloat16)
    y_ref[...] = lax.dot_general(
        h, w2, (((1,), (0,)), ((), ())),
        preferred_element_type=jnp.float32,
    )


def _gmm(bexp, xs, gs, w1, w2):
    grid_spec = pltpu.PrefetchScalarGridSpec(
        num_scalar_prefetch=1,
        grid=(_NBLK,),
        in_specs=[
            pl.BlockSpec((_M, _D), lambda b, be: (b, 0)),
            pl.BlockSpec((_M, 128), lambda b, be: (b, 0)),
            pl.BlockSpec((1, _D, _F), lambda b, be: (be[b], 0, 0)),
            pl.BlockSpec((1, _F, _D), lambda b, be: (be[b], 0, 0)),
        ],
        out_specs=pl.BlockSpec((_M, _D), lambda b, be: (b, 0)),
    )
    return pl.pallas_call(
        _gmm_body,
        grid_spec=grid_spec,
        out_shape=jax.ShapeDtypeStruct((_CAP, _D), jnp.float32),
    )(bexp, xs, gs, w1, w2)


_NC, _NS = 2, 16
_NW = _NC * _NS
_TPW = _T // _NW


def _sc_scatter(x2d, ge, go, pos_e, pos_o):
    mesh = plsc.VectorSubcoreMesh(core_axis_name="c", subcore_axis_name="s")

    @functools.partial(
        pl.kernel, mesh=mesh,
        out_type=(
            jax.ShapeDtypeStruct((_CAP, _D), jnp.float32),
            jax.ShapeDtypeStruct((_CAP, 128), jnp.float32),
        ),
        scratch_types=[
            pltpu.VMEM((_TPW,), jnp.int32),
            pltpu.VMEM((_TPW,), jnp.int32),
            pltpu.VMEM((_TPW, _D), jnp.float32),
            pltpu.VMEM((_TPW, 128), jnp.float32),
            pltpu.VMEM((_TPW, 128), jnp.float32),
            pltpu.SemaphoreType.DMA,
            pltpu.SemaphoreType.DMA,
            pltpu.SemaphoreType.DMA,
            pltpu.SemaphoreType.DMA,
        ],
    )
    def k(x_hbm, ge_hbm, go_hbm, pe_hbm, po_hbm, xs_hbm, gs_hbm,
          pev, pov, xv, gev, gov, s1, s2, s3, s4):
        wid = lax.axis_index("s") * _NC + lax.axis_index("c")
        base = wid * _TPW
        pltpu.sync_copy(pe_hbm.at[pl.ds(base, _TPW)], pev)
        pltpu.sync_copy(po_hbm.at[pl.ds(base, _TPW)], pov)
        pltpu.sync_copy(x_hbm.at[pl.ds(base, _TPW)], xv)
        pltpu.sync_copy(ge_hbm.at[pl.ds(base, _TPW)], gev)
        pltpu.sync_copy(go_hbm.at[pl.ds(base, _TPW)], gov)
        c1 = pltpu.async_copy(xv, xs_hbm.at[pev], s1)
        c2 = pltpu.async_copy(xv, xs_hbm.at[pov], s2)
        c3 = pltpu.async_copy(gev, gs_hbm.at[pev], s3)
        c4 = pltpu.async_copy(gov, gs_hbm.at[pov], s4)
        c1.wait()
        c2.wait()
        c3.wait()
        c4.wait()

    return k(x2d, ge, go, pos_e, pos_o)


def _sc_combine(y, pos_e, pos_o):
    mesh = plsc.VectorSubcoreMesh(core_axis_name="c", subcore_axis_name="s")
    half = _TPW // 2

    @functools.partial(
        pl.kernel, mesh=mesh,
        out_type=jax.ShapeDtypeStruct((_T, _D), jnp.float32),
        scratch_types=[
            pltpu.VMEM((_TPW,), jnp.int32),
            pltpu.VMEM((_TPW,), jnp.int32),
            pltpu.VMEM((half, _D), jnp.float32),
            pltpu.VMEM((half, _D), jnp.float32),
            pltpu.SemaphoreType.DMA,
            pltpu.SemaphoreType.DMA,
        ],
    )
    def k(y_hbm, pe_hbm, po_hbm, out_hbm, pev, pov, rowsa, rowsb, sem1, sem2):
        wid = lax.axis_index("s") * _NC + lax.axis_index("c")
        base = wid * _TPW
        pltpu.sync_copy(pe_hbm.at[pl.ds(base, _TPW)], pev)
        pltpu.sync_copy(po_hbm.at[pl.ds(base, _TPW)], pov)
        for c in range(_TPW // half):
            ca = pltpu.async_copy(
                y_hbm.at[pev.at[pl.ds(half * c, half)]], rowsa, sem1)
            cb = pltpu.async_copy(
                y_hbm.at[pov.at[pl.ds(half * c, half)]], rowsb, sem2)
            ca.wait()
            cb.wait()

            def body(j, _):
                for l in range(_D // 16):
                    a = rowsa[j, pl.ds(16 * l, 16)]
                    b = rowsb[j, pl.ds(16 * l, 16)]
                    rowsa[j, pl.ds(16 * l, 16)] = a + b
                return 0

            lax.fori_loop(0, half, body, 0)
            pltpu.sync_copy(rowsa, out_hbm.at[pl.ds(base + half * c, half)])

    return k(y, pos_e, pos_o)


def _sc_combine_reduce(y, pos_e, pos_o):
    mesh = plsc.VectorSubcoreMesh(core_axis_name="c", subcore_axis_name="s")
    half = _TPW // 2

    @functools.partial(
        pl.kernel, mesh=mesh,
        out_type=jax.ShapeDtypeStruct((_NW, _D), jnp.float32),
        scratch_types=[
            pltpu.VMEM((_TPW,), jnp.int32),
            pltpu.VMEM((_TPW,), jnp.int32),
            pltpu.VMEM((half, _D), jnp.float32),
            pltpu.VMEM((half, _D), jnp.float32),
            pltpu.VMEM((1, _D), jnp.float32),
            pltpu.SemaphoreType.DMA,
            pltpu.SemaphoreType.DMA,
        ],
    )
    def k(y_hbm, pe_hbm, po_hbm, out_hbm, pev, pov, rowsa, rowsb, acc,
          sem1, sem2):
        wid = lax.axis_index("s") * _NC + lax.axis_index("c")
        base = wid * _TPW
        pltpu.sync_copy(pe_hbm.at[pl.ds(base, _TPW)], pev)
        pltpu.sync_copy(po_hbm.at[pl.ds(base, _TPW)], pov)
        zero = jnp.zeros((16,), jnp.float32)
        for l in range(_D // 16):
            acc[0, pl.ds(16 * l, 16)] = zero
        for c in range(_TPW // half):
            ca = pltpu.async_copy(
                y_hbm.at[pev.at[pl.ds(half * c, half)]], rowsa, sem1)
            cb = pltpu.async_copy(
                y_hbm.at[pov.at[pl.ds(half * c, half)]], rowsb, sem2)
            ca.wait()
            cb.wait()

            def body(j, _):
                for l in range(_D // 16):
                    a = rowsa[j, pl.ds(16 * l, 16)]
                    b = rowsb[j, pl.ds(16 * l, 16)]
                    acc[0, pl.ds(16 * l, 16)] = acc[0, pl.ds(16 * l, 16)] + a + b
                return 0

            lax.fori_loop(0, half, body, 0)
        pltpu.sync_copy(acc, out_hbm.at[pl.ds(wid, 1)])

    return k(y, pos_e, pos_o)


def _loss_body(y_ref, xsum_ref, part_ref, out_ref):
    sent = (xsum_ref[...] + jnp.sum(part_ref[...], axis=0, keepdims=True))
    sent = sent / float(_T)
    mx = jnp.max(sent)
    lse = jnp.log(jnp.sum(jnp.exp(sent - mx))) + mx
    yv = y_ref[0]
    lane = lax.broadcasted_iota(jnp.int32, sent.shape, 1)
    picked = jnp.sum(jnp.where(lane == yv, sent, 0.0))
    out_ref[0, 0] = lse - picked


def _loss(y, xsum, partials):
    return pl.pallas_call(
        _loss_body,
        in_specs=[
            pl.BlockSpec(memory_space=pltpu.SMEM),
            pl.BlockSpec((1, _D), lambda: (0, 0)),
            pl.BlockSpec((_NW, _D), lambda: (0, 0)),
        ],
        out_specs=pl.BlockSpec(memory_space=pltpu.SMEM),
        out_shape=jax.ShapeDtypeStruct((1, 1), jnp.float32),
    )(y, xsum, partials)


def kernel(x, y, Wg1, W1a, W1b, Wg2, W2a, W2b):
    xt = x.reshape(_T, _D)
    ge1, go1, pos1, bexp1, xsum = _router(xt, Wg1)
    pe1, po1 = pos1[:, 0], pos1[:, 1]
    xs1, gs1 = _sc_scatter(xt, ge1, go1, pe1, po1)
    y1 = _gmm(bexp1.reshape(-1), xs1, gs1, W1a, W1b)
    m1 = _sc_combine(y1, pe1, po1)
    ge2, go2, pos2, bexp2, _ = _router(m1, Wg2)
    pe2, po2 = pos2[:, 0], pos2[:, 1]
    xs2, gs2 = _sc_scatter(m1, ge2, go2, pe2, po2)
    y2 = _gmm(bexp2.reshape(-1), xs2, gs2, W2a, W2b)
    part = _sc_combine_reduce(y2, pe2, po2)
    out = _loss(y.astype(jnp.int32), xsum, part)
    return out[0, 0]


# speedup vs baseline: 1.0845x; 1.0222x over previous
import functools

import jax
import jax.numpy as jnp
from jax import lax
from jax.experimental import pallas as pl
from jax.experimental.pallas import tpu as pltpu
from jax.experimental.pallas import tpu_sc as plsc

_T = 2048
_D = 1024
_F = 1024
_E = 8
_M = 256
_CAP = 4096 + _E * _M
_NBLK = _CAP // _M


def _router_body(x_ref, wg_ref, ge_ref, go_ref, pos_ref, bexp_ref, xsum_ref):
    x = x_ref[...]
    wg = wg_ref[...]
    logits = lax.dot_general(
        x, wg, (((1,), (0,)), ((), ())),
        preferred_element_type=jnp.float32,
        precision=lax.Precision.HIGHEST,
    )
    m = jnp.max(logits, axis=1, keepdims=True)
    p = jnp.exp(logits - m)
    p = p / jnp.sum(p, axis=1, keepdims=True)
    lane = lax.broadcasted_iota(jnp.int32, p.shape, 1)
    m1 = jnp.max(p, axis=1, keepdims=True)
    i1 = jnp.min(jnp.where(p == m1, lane, _E), axis=1, keepdims=True)
    p2 = jnp.where(lane == i1, -1.0, p)
    m2 = jnp.max(p2, axis=1, keepdims=True)
    i2 = jnp.min(jnp.where(p2 == m2, lane, _E), axis=1, keepdims=True)
    den = m1 + m2 + 1e-9
    g1 = m1 / den
    g2 = m2 / den
    ge_ref[...] = jnp.broadcast_to(g1, (_T, 128))
    go_ref[...] = jnp.broadcast_to(g2, (_T, 128))
    xsum_ref[...] = jnp.sum(x, axis=0, keepdims=True)

    onehot = (jnp.where(lane == i1, 1.0, 0.0)
              + jnp.where(lane == i2, 1.0, 0.0)).astype(jnp.bfloat16)
    ri = lax.broadcasted_iota(jnp.int32, (256, 256), 0)
    ci = lax.broadcasted_iota(jnp.int32, (256, 256), 1)
    ltri = (ci < ri).astype(jnp.bfloat16)
    parts = []
    carry = jnp.zeros((1, _E), jnp.float32)
    for j in range(_T // 256):
        oh = onehot[j * 256:(j + 1) * 256]
        r = lax.dot_general(
            ltri, oh, (((1,), (0,)), ((), ())),
            preferred_element_type=jnp.float32,
        ) + carry
        parts.append(r)
        carry = carry + jnp.sum(oh.astype(jnp.float32), axis=0, keepdims=True)
    ranks = jnp.concatenate(parts, axis=0)
    counts = carry
    pt = jnp.ceil(counts / _M) * _M
    r8 = lax.broadcasted_iota(jnp.int32, (_E, _E), 0)
    c8 = lax.broadcasted_iota(jnp.int32, (_E, _E), 1)
    utri = (r8 < c8).astype(jnp.float32)
    off = lax.dot_general(
        pt, utri, (((1,), (0,)), ((), ())),
        preferred_element_type=jnp.float32,
        precision=lax.Precision.HIGHEST,
    )
    off_b = jnp.broadcast_to(off, (_T, _E))
    rank1 = jnp.sum(jnp.where(lane == i1, ranks, 0.0), axis=1, keepdims=True)
    rank2 = jnp.sum(jnp.where(lane == i2, ranks, 0.0), axis=1, keepdims=True)
    off1 = jnp.sum(jnp.where(lane == i1, off_b, 0.0), axis=1, keepdims=True)
    off2 = jnp.sum(jnp.where(lane == i2, off_b, 0.0), axis=1, keepdims=True)
    pos1 = (off1 + rank1).astype(jnp.int32)
    pos2 = (off2 + rank2).astype(jnp.int32)
    pos_ref[...] = jnp.concatenate([pos1, pos2], axis=1)

    brow = lax.broadcasted_iota(jnp.int32, (_NBLK, _E), 0).astype(jnp.float32) * _M
    blane = lax.broadcasted_iota(jnp.int32, (_NBLK, _E), 1)
    off_nb = jnp.broadcast_to(off, (_NBLK, _E))
    pt_nb = jnp.broadcast_to(pt, (_NBLK, _E))
    member = jnp.logical_and(brow >= off_nb, brow < off_nb + pt_nb)
    bexp_ref[...] = jnp.sum(jnp.where(member, blane, 0), axis=1, keepdims=True)


def _router(xt, wg):
    return pl.pallas_call(
        _router_body,
        out_shape=(
            jax.ShapeDtypeStruct((_T, 128), jnp.float32),
            jax.ShapeDtypeStruct((_T, 128), jnp.float32),
            jax.ShapeDtypeStruct((_T, 2), jnp.int32),
            jax.ShapeDtypeStruct((_NBLK, 1), jnp.int32),
            jax.ShapeDtypeStruct((1, _D), jnp.float32),
        ),
    )(xt, wg)


def _gmm_body(bexp_ref, xs_ref, gs_ref, w1_ref, w2_ref, y_ref):
    xb = xs_ref[...].astype(jnp.bfloat16)
    w1 = w1_ref[0].astype(jnp.bfloat16)
    w2 = w2_ref[0].astype(jnp.bfloat16)
    h = lax.dot_general(
        xb, w1, (((1,), (0,)), ((), ())),
        preferred_element_type=jnp.float32,
    )
    g = gs_ref[...][:, 0:1]
    h = (jnp.maximum(h, 0.0) * g).astype(jnp.bfloat16)
    y_ref[...] = lax.dot_general(
        h, w2, (((1,), (0,)), ((), ())),
        preferred_element_type=jnp.float32,
    )


def _gmm(bexp, xs, gs, w1, w2):
    grid_spec = pltpu.PrefetchScalarGridSpec(
        num_scalar_prefetch=1,
        grid=(_NBLK,),
        in_specs=[
            pl.BlockSpec((_M, _D), lambda b, be: (b, 0)),
            pl.BlockSpec((_M, 128), lambda b, be: (b, 0)),
            pl.BlockSpec((1, _D, _F), lambda b, be: (be[b], 0, 0)),
            pl.BlockSpec((1, _F, _D), lambda b, be: (be[b], 0, 0)),
        ],
        out_specs=pl.BlockSpec((_M, _D), lambda b, be: (b, 0)),
    )
    return pl.pallas_call(
        _gmm_body,
        grid_spec=grid_spec,
        out_shape=jax.ShapeDtypeStruct((_CAP, _D), jnp.float32),
    )(bexp, xs, gs, w1, w2)


_NC, _NS = 2, 16
_NW = _NC * _NS
_TPW = _T // _NW


def _sc_scatter(x2d, ge, go, pos_e, pos_o):
    mesh = plsc.VectorSubcoreMesh(core_axis_name="c", subcore_axis_name="s")

    @functools.partial(
        pl.kernel, mesh=mesh,
        out_type=(
            jax.ShapeDtypeStruct((_CAP, _D), jnp.float32),
            jax.ShapeDtypeStruct((_CAP, 128), jnp.float32),
        ),
        scratch_types=[
            pltpu.VMEM((_TPW,), jnp.int32),
            pltpu.VMEM((_TPW,), jnp.int32),
            pltpu.VMEM((_TPW, _D), jnp.float32),
            pltpu.VMEM((_TPW, 128), jnp.float32),
            pltpu.VMEM((_TPW, 128), jnp.float32),
            pltpu.SemaphoreType.DMA,
            pltpu.SemaphoreType.DMA,
            pltpu.SemaphoreType.DMA,
            pltpu.SemaphoreType.DMA,
        ],
    )
    def k(x_hbm, ge_hbm, go_hbm, pe_hbm, po_hbm, xs_hbm, gs_hbm,
          pev, pov, xv, gev, gov, s1, s2, s3, s4):
        wid = lax.axis_index("s") * _NC + lax.axis_index("c")
        base = wid * _TPW
        pltpu.sync_copy(pe_hbm.at[pl.ds(base, _TPW)], pev)
        pltpu.sync_copy(po_hbm.at[pl.ds(base, _TPW)], pov)
        pltpu.sync_copy(x_hbm.at[pl.ds(base, _TPW)], xv)
        pltpu.sync_copy(ge_hbm.at[pl.ds(base, _TPW)], gev)
        pltpu.sync_copy(go_hbm.at[pl.ds(base, _TPW)], gov)
        c1 = pltpu.async_copy(xv, xs_hbm.at[pev], s1)
        c2 = pltpu.async_copy(xv, xs_hbm.at[pov], s2)
        c3 = pltpu.async_copy(gev, gs_hbm.at[pev], s3)
        c4 = pltpu.async_copy(gov, gs_hbm.at[pov], s4)
        c1.wait()
        c2.wait()
        c3.wait()
        c4.wait()

    return k(x2d, ge, go, pos_e, pos_o)


def _sc_combine(y, pos_e, pos_o):
    mesh = plsc.VectorSubcoreMesh(core_axis_name="c", subcore_axis_name="s")
    half = _TPW // 2

    @functools.partial(
        pl.kernel, mesh=mesh,
        out_type=jax.ShapeDtypeStruct((_T, _D), jnp.float32),
        scratch_types=[
            pltpu.VMEM((_TPW,), jnp.int32),
            pltpu.VMEM((_TPW,), jnp.int32),
            pltpu.VMEM((half, _D), jnp.float32),
            pltpu.VMEM((half, _D), jnp.float32),
            pltpu.SemaphoreType.DMA,
            pltpu.SemaphoreType.DMA,
        ],
    )
    def k(y_hbm, pe_hbm, po_hbm, out_hbm, pev, pov, rowsa, rowsb, sem1, sem2):
        wid = lax.axis_index("s") * _NC + lax.axis_index("c")
        base = wid * _TPW
        pltpu.sync_copy(pe_hbm.at[pl.ds(base, _TPW)], pev)
        pltpu.sync_copy(po_hbm.at[pl.ds(base, _TPW)], pov)
        for c in range(_TPW // half):
            ca = pltpu.async_copy(
                y_hbm.at[pev.at[pl.ds(half * c, half)]], rowsa, sem1)
            cb = pltpu.async_copy(
                y_hbm.at[pov.at[pl.ds(half * c, half)]], rowsb, sem2)
            ca.wait()
            cb.wait()

            def body(j, _):
                for l in range(_D // 16):
                    a = rowsa[j, pl.ds(16 * l, 16)]
                    b = rowsb[j, pl.ds(16 * l, 16)]
                    rowsa[j, pl.ds(16 * l, 16)] = a + b
                return 0

            lax.fori_loop(0, half, body, 0)
            pltpu.sync_copy(rowsa, out_hbm.at[pl.ds(base + half * c, half)])

    return k(y, pos_e, pos_o)


def _sc_combine_reduce(y, pos_e, pos_o):
    mesh = plsc.VectorSubcoreMesh(core_axis_name="c", subcore_axis_name="s")
    half = _TPW // 2

    @functools.partial(
        pl.kernel, mesh=mesh,
        out_type=jax.ShapeDtypeStruct((_NW, _D), jnp.float32),
        scratch_types=[
            pltpu.VMEM((_TPW,), jnp.int32),
            pltpu.VMEM((_TPW,), jnp.int32),
            pltpu.VMEM((half, _D), jnp.float32),
            pltpu.VMEM((half, _D), jnp.float32),
            pltpu.VMEM((1, _D), jnp.float32),
            pltpu.SemaphoreType.DMA,
            pltpu.SemaphoreType.DMA,
        ],
    )
    def k(y_hbm, pe_hbm, po_hbm, out_hbm, pev, pov, rowsa, rowsb, acc,
          sem1, sem2):
        wid = lax.axis_index("s") * _NC + lax.axis_index("c")
        base = wid * _TPW
        pltpu.sync_copy(pe_hbm.at[pl.ds(base, _TPW)], pev)
        pltpu.sync_copy(po_hbm.at[pl.ds(base, _TPW)], pov)
        zero = jnp.zeros((16,), jnp.float32)
        for l in range(_D // 16):
            acc[0, pl.ds(16 * l, 16)] = zero
        for c in range(_TPW // half):
            ca = pltpu.async_copy(
                y_hbm.at[pev.at[pl.ds(half * c, half)]], rowsa, sem1)
            cb = pltpu.async_copy(
                y_hbm.at[pov.at[pl.ds(half * c, half)]], rowsb, sem2)
            ca.wait()
            cb.wait()

            def body(j, _):
                for l in range(_D // 16):
                    a = rowsa[j, pl.ds(16 * l, 16)]
                    b = rowsb[j, pl.ds(16 * l, 16)]
                    rowsa[j, pl.ds(16 * l, 16)] = a + b
                return 0

            lax.fori_loop(0, half, body, 0)
            for s in (16, 8, 4, 2, 1):

                def tbody(j, _, s=s):
                    for l in range(_D // 16):
                        rowsa[j, pl.ds(16 * l, 16)] = (
                            rowsa[j, pl.ds(16 * l, 16)]
                            + rowsa[j + s, pl.ds(16 * l, 16)])
                    return 0

                lax.fori_loop(0, s, tbody, 0)
            if True:
                for l in range(_D // 16):
                    acc[0, pl.ds(16 * l, 16)] = (
                        acc[0, pl.ds(16 * l, 16)] + rowsa[0, pl.ds(16 * l, 16)])
        pltpu.sync_copy(acc, out_hbm.at[pl.ds(wid, 1)])

    return k(y, pos_e, pos_o)


def _loss_body(y_ref, xsum_ref, part_ref, out_ref):
    sent = (xsum_ref[...] + jnp.sum(part_ref[...], axis=0, keepdims=True))
    sent = sent / float(_T)
    mx = jnp.max(sent)
    lse = jnp.log(jnp.sum(jnp.exp(sent - mx))) + mx
    yv = y_ref[0]
    lane = lax.broadcasted_iota(jnp.int32, sent.shape, 1)
    picked = jnp.sum(jnp.where(lane == yv, sent, 0.0))
    out_ref[0, 0] = lse - picked


def _loss(y, xsum, partials):
    return pl.pallas_call(
        _loss_body,
        in_specs=[
            pl.BlockSpec(memory_space=pltpu.SMEM),
            pl.BlockSpec((1, _D), lambda: (0, 0)),
            pl.BlockSpec((_NW, _D), lambda: (0, 0)),
        ],
        out_specs=pl.BlockSpec(memory_space=pltpu.SMEM),
        out_shape=jax.ShapeDtypeStruct((1, 1), jnp.float32),
    )(y, xsum, partials)


def kernel(x, y, Wg1, W1a, W1b, Wg2, W2a, W2b):
    xt = x.reshape(_T, _D)
    ge1, go1, pos1, bexp1, xsum = _router(xt, Wg1)
    pe1, po1 = pos1[:, 0], pos1[:, 1]
    xs1, gs1 = _sc_scatter(xt, ge1, go1, pe1, po1)
    y1 = _gmm(bexp1.reshape(-1), xs1, gs1, W1a, W1b)
    m1 = _sc_combine(y1, pe1, po1)
    ge2, go2, pos2, bexp2, _ = _router(m1, Wg2)
    pe2, po2 = pos2[:, 0], pos2[:, 1]
    xs2, gs2 = _sc_scatter(m1, ge2, go2, pe2, po2)
    y2 = _gmm(bexp2.reshape(-1), xs2, gs2, W2a, W2b)
    part = _sc_combine_reduce(y2, pe2, po2)
    out = _loss(y.astype(jnp.int32), xsum, part)
    return out[0, 0]


# speedup vs baseline: 1.1329x; 1.0446x over previous
import functools

import jax
import jax.numpy as jnp
from jax import lax
from jax.experimental import pallas as pl
from jax.experimental.pallas import tpu as pltpu
from jax.experimental.pallas import tpu_sc as plsc

_T = 2048
_D = 1024
_F = 1024
_E = 8
_M = 256
_CAP = 4096 + _E * _M
_NBLK = _CAP // _M


def _router_body(x_ref, wg_ref, ge_ref, go_ref, pos_ref, bexp_ref, xsum_ref):
    x = x_ref[...]
    wg = wg_ref[...]
    logits = lax.dot_general(
        x.astype(jnp.bfloat16), wg.astype(jnp.bfloat16),
        (((1,), (0,)), ((), ())),
        preferred_element_type=jnp.float32,
    )
    m = jnp.max(logits, axis=1, keepdims=True)
    p = jnp.exp(logits - m)
    p = p / jnp.sum(p, axis=1, keepdims=True)
    lane = lax.broadcasted_iota(jnp.int32, p.shape, 1)
    m1 = jnp.max(p, axis=1, keepdims=True)
    i1 = jnp.min(jnp.where(p == m1, lane, _E), axis=1, keepdims=True)
    p2 = jnp.where(lane == i1, -1.0, p)
    m2 = jnp.max(p2, axis=1, keepdims=True)
    i2 = jnp.min(jnp.where(p2 == m2, lane, _E), axis=1, keepdims=True)
    den = m1 + m2 + 1e-9
    g1 = m1 / den
    g2 = m2 / den
    ge_ref[...] = jnp.broadcast_to(g1, (_T, 128))
    go_ref[...] = jnp.broadcast_to(g2, (_T, 128))
    xsum_ref[...] = jnp.sum(x, axis=0, keepdims=True)

    onehot = (jnp.where(lane == i1, 1.0, 0.0)
              + jnp.where(lane == i2, 1.0, 0.0)).astype(jnp.bfloat16)
    ri = lax.broadcasted_iota(jnp.int32, (256, 256), 0)
    ci = lax.broadcasted_iota(jnp.int32, (256, 256), 1)
    ltri = (ci < ri).astype(jnp.bfloat16)
    parts = []
    carry = jnp.zeros((1, _E), jnp.float32)
    for j in range(_T // 256):
        oh = onehot[j * 256:(j + 1) * 256]
        r = lax.dot_general(
            ltri, oh, (((1,), (0,)), ((), ())),
            preferred_element_type=jnp.float32,
        ) + carry
        parts.append(r)
        carry = carry + jnp.sum(oh.astype(jnp.float32), axis=0, keepdims=True)
    ranks = jnp.concatenate(parts, axis=0)
    counts = carry
    pt = jnp.ceil(counts / _M) * _M
    r8 = lax.broadcasted_iota(jnp.int32, (_E, _E), 0)
    c8 = lax.broadcasted_iota(jnp.int32, (_E, _E), 1)
    utri = (r8 < c8).astype(jnp.float32)
    off = lax.dot_general(
        pt, utri, (((1,), (0,)), ((), ())),
        preferred_element_type=jnp.float32,
        precision=lax.Precision.HIGHEST,
    )
    off_b = jnp.broadcast_to(off, (_T, _E))
    rank1 = jnp.sum(jnp.where(lane == i1, ranks, 0.0), axis=1, keepdims=True)
    rank2 = jnp.sum(jnp.where(lane == i2, ranks, 0.0), axis=1, keepdims=True)
    off1 = jnp.sum(jnp.where(lane == i1, off_b, 0.0), axis=1, keepdims=True)
    off2 = jnp.sum(jnp.where(lane == i2, off_b, 0.0), axis=1, keepdims=True)
    pos1 = (off1 + rank1).astype(jnp.int32)
    pos2 = (off2 + rank2).astype(jnp.int32)
    pos_ref[...] = jnp.concatenate([pos1, pos2], axis=1)

    brow = lax.broadcasted_iota(jnp.int32, (_NBLK, _E), 0).astype(jnp.float32) * _M
    blane = lax.broadcasted_iota(jnp.int32, (_NBLK, _E), 1)
    off_nb = jnp.broadcast_to(off, (_NBLK, _E))
    pt_nb = jnp.broadcast_to(pt, (_NBLK, _E))
    member = jnp.logical_and(brow >= off_nb, brow < off_nb + pt_nb)
    bexp_ref[...] = jnp.sum(jnp.where(member, blane, 0), axis=1, keepdims=True)


def _router(xt, wg):
    return pl.pallas_call(
        _router_body,
        out_shape=(
            jax.ShapeDtypeStruct((_T, 128), jnp.float32),
            jax.ShapeDtypeStruct((_T, 128), jnp.float32),
            jax.ShapeDtypeStruct((_T, 2), jnp.int32),
            jax.ShapeDtypeStruct((_NBLK, 1), jnp.int32),
            jax.ShapeDtypeStruct((1, _D), jnp.float32),
        ),
    )(xt, wg)


def _gmm_body(bexp_ref, xs_ref, gs_ref, w1_ref, w2_ref, y_ref):
    xb = xs_ref[...].astype(jnp.bfloat16)
    w1 = w1_ref[0].astype(jnp.bfloat16)
    w2 = w2_ref[0].astype(jnp.bfloat16)
    h = lax.dot_general(
        xb, w1, (((1,), (0,)), ((), ())),
        preferred_element_type=jnp.float32,
    )
    g = gs_ref[...][:, 0:1]
    h = (jnp.maximum(h, 0.0) * g).astype(jnp.bfloat16)
    y_ref[...] = lax.dot_general(
        h, w2, (((1,), (0,)), ((), ())),
        preferred_element_type=jnp.float32,
    )


def _gmm(bexp, xs, gs, w1, w2):
    grid_spec = pltpu.PrefetchScalarGridSpec(
        num_scalar_prefetch=1,
        grid=(_NBLK,),
        in_specs=[
            pl.BlockSpec((_M, _D), lambda b, be: (b, 0)),
            pl.BlockSpec((_M, 128), lambda b, be: (b, 0)),
            pl.BlockSpec((1, _D, _F), lambda b, be: (be[b], 0, 0)),
            pl.BlockSpec((1, _F, _D), lambda b, be: (be[b], 0, 0)),
        ],
        out_specs=pl.BlockSpec((_M, _D), lambda b, be: (b, 0)),
    )
    return pl.pallas_call(
        _gmm_body,
        grid_spec=grid_spec,
        out_shape=jax.ShapeDtypeStruct((_CAP, _D), jnp.float32),
    )(bexp, xs, gs, w1, w2)


_NC, _NS = 2, 16
_NW = _NC * _NS
_TPW = _T // _NW


def _sc_scatter(x2d, ge, go, pos_e, pos_o):
    mesh = plsc.VectorSubcoreMesh(core_axis_name="c", subcore_axis_name="s")

    @functools.partial(
        pl.kernel, mesh=mesh,
        out_type=(
            jax.ShapeDtypeStruct((_CAP, _D), jnp.float32),
            jax.ShapeDtypeStruct((_CAP, 128), jnp.float32),
        ),
        scratch_types=[
            pltpu.VMEM((_TPW,), jnp.int32),
            pltpu.VMEM((_TPW,), jnp.int32),
            pltpu.VMEM((_TPW, _D), jnp.float32),
            pltpu.VMEM((_TPW, 128), jnp.float32),
            pltpu.VMEM((_TPW, 128), jnp.float32),
            pltpu.SemaphoreType.DMA,
            pltpu.SemaphoreType.DMA,
            pltpu.SemaphoreType.DMA,
            pltpu.SemaphoreType.DMA,
        ],
    )
    def k(x_hbm, ge_hbm, go_hbm, pe_hbm, po_hbm, xs_hbm, gs_hbm,
          pev, pov, xv, gev, gov, s1, s2, s3, s4):
        wid = lax.axis_index("s") * _NC + lax.axis_index("c")
        base = wid * _TPW
        pltpu.sync_copy(pe_hbm.at[pl.ds(base, _TPW)], pev)
        pltpu.sync_copy(po_hbm.at[pl.ds(base, _TPW)], pov)
        pltpu.sync_copy(x_hbm.at[pl.ds(base, _TPW)], xv)
        pltpu.sync_copy(ge_hbm.at[pl.ds(base, _TPW)], gev)
        pltpu.sync_copy(go_hbm.at[pl.ds(base, _TPW)], gov)
        c1 = pltpu.async_copy(xv, xs_hbm.at[pev], s1)
        c2 = pltpu.async_copy(xv, xs_hbm.at[pov], s2)
        c3 = pltpu.async_copy(gev, gs_hbm.at[pev], s3)
        c4 = pltpu.async_copy(gov, gs_hbm.at[pov], s4)
        c1.wait()
        c2.wait()
        c3.wait()
        c4.wait()

    return k(x2d, ge, go, pos_e, pos_o)


def _sc_combine(y, pos_e, pos_o):
    mesh = plsc.VectorSubcoreMesh(core_axis_name="c", subcore_axis_name="s")
    half = _TPW // 2

    @functools.partial(
        pl.kernel, mesh=mesh,
        out_type=jax.ShapeDtypeStruct((_T, _D), jnp.float32),
        scratch_types=[
            pltpu.VMEM((_TPW,), jnp.int32),
            pltpu.VMEM((_TPW,), jnp.int32),
            pltpu.VMEM((half, _D), jnp.float32),
            pltpu.VMEM((half, _D), jnp.float32),
            pltpu.SemaphoreType.DMA,
            pltpu.SemaphoreType.DMA,
        ],
    )
    def k(y_hbm, pe_hbm, po_hbm, out_hbm, pev, pov, rowsa, rowsb, sem1, sem2):
        wid = lax.axis_index("s") * _NC + lax.axis_index("c")
        base = wid * _TPW
        pltpu.sync_copy(pe_hbm.at[pl.ds(base, _TPW)], pev)
        pltpu.sync_copy(po_hbm.at[pl.ds(base, _TPW)], pov)
        for c in range(_TPW // half):
            ca = pltpu.async_copy(
                y_hbm.at[pev.at[pl.ds(half * c, half)]], rowsa, sem1)
            cb = pltpu.async_copy(
                y_hbm.at[pov.at[pl.ds(half * c, half)]], rowsb, sem2)
            ca.wait()
            cb.wait()

            def body(j, _):
                for l in range(_D // 16):
                    a = rowsa[j, pl.ds(16 * l, 16)]
                    b = rowsb[j, pl.ds(16 * l, 16)]
                    rowsa[j, pl.ds(16 * l, 16)] = a + b
                return 0

            lax.fori_loop(0, half, body, 0)
            pltpu.sync_copy(rowsa, out_hbm.at[pl.ds(base + half * c, half)])

    return k(y, pos_e, pos_o)


def _sc_combine_reduce(y, pos_e, pos_o):
    mesh = plsc.VectorSubcoreMesh(core_axis_name="c", subcore_axis_name="s")
    half = _TPW // 2

    @functools.partial(
        pl.kernel, mesh=mesh,
        out_type=jax.ShapeDtypeStruct((_NW, _D), jnp.float32),
        scratch_types=[
            pltpu.VMEM((_TPW,), jnp.int32),
            pltpu.VMEM((_TPW,), jnp.int32),
            pltpu.VMEM((half, _D), jnp.float32),
            pltpu.VMEM((half, _D), jnp.float32),
            pltpu.VMEM((1, _D), jnp.float32),
            pltpu.SemaphoreType.DMA,
            pltpu.SemaphoreType.DMA,
        ],
    )
    def k(y_hbm, pe_hbm, po_hbm, out_hbm, pev, pov, rowsa, rowsb, acc,
          sem1, sem2):
        wid = lax.axis_index("s") * _NC + lax.axis_index("c")
        base = wid * _TPW
        pltpu.sync_copy(pe_hbm.at[pl.ds(base, _TPW)], pev)
        pltpu.sync_copy(po_hbm.at[pl.ds(base, _TPW)], pov)
        zero = jnp.zeros((16,), jnp.float32)
        for l in range(_D // 16):
            acc[0, pl.ds(16 * l, 16)] = zero
        for c in range(_TPW // half):
            ca = pltpu.async_copy(
                y_hbm.at[pev.at[pl.ds(half * c, half)]], rowsa, sem1)
            cb = pltpu.async_copy(
                y_hbm.at[pov.at[pl.ds(half * c, half)]], rowsb, sem2)
            ca.wait()
            cb.wait()

            def body(j, _):
                for l in range(_D // 16):
                    a = rowsa[j, pl.ds(16 * l, 16)]
                    b = rowsb[j, pl.ds(16 * l, 16)]
                    rowsa[j, pl.ds(16 * l, 16)] = a + b
                return 0

            lax.fori_loop(0, half, body, 0)
            for s in (16, 8, 4, 2, 1):

                def tbody(j, _, s=s):
                    for l in range(_D // 16):
                        rowsa[j, pl.ds(16 * l, 16)] = (
                            rowsa[j, pl.ds(16 * l, 16)]
                            + rowsa[j + s, pl.ds(16 * l, 16)])
                    return 0

                lax.fori_loop(0, s, tbody, 0)
            if True:
                for l in range(_D // 16):
                    acc[0, pl.ds(16 * l, 16)] = (
                        acc[0, pl.ds(16 * l, 16)] + rowsa[0, pl.ds(16 * l, 16)])
        pltpu.sync_copy(acc, out_hbm.at[pl.ds(wid, 1)])

    return k(y, pos_e, pos_o)


def _loss_body(y_ref, xsum_ref, part_ref, out_ref):
    sent = (xsum_ref[...] + jnp.sum(part_ref[...], axis=0, keepdims=True))
    sent = sent / float(_T)
    mx = jnp.max(sent)
    lse = jnp.log(jnp.sum(jnp.exp(sent - mx))) + mx
    yv = y_ref[0]
    lane = lax.broadcasted_iota(jnp.int32, sent.shape, 1)
    picked = jnp.sum(jnp.where(lane == yv, sent, 0.0))
    out_ref[0, 0] = lse - picked


def _loss(y, xsum, partials):
    return pl.pallas_call(
        _loss_body,
        in_specs=[
            pl.BlockSpec(memory_space=pltpu.SMEM),
            pl.BlockSpec((1, _D), lambda: (0, 0)),
            pl.BlockSpec((_NW, _D), lambda: (0, 0)),
        ],
        out_specs=pl.BlockSpec(memory_space=pltpu.SMEM),
        out_shape=jax.ShapeDtypeStruct((1, 1), jnp.float32),
    )(y, xsum, partials)


def kernel(x, y, Wg1, W1a, W1b, Wg2, W2a, W2b):
    xt = x.reshape(_T, _D)
    ge1, go1, pos1, bexp1, xsum = _router(xt, Wg1)
    pe1, po1 = pos1[:, 0], pos1[:, 1]
    xs1, gs1 = _sc_scatter(xt, ge1, go1, pe1, po1)
    y1 = _gmm(bexp1.reshape(-1), xs1, gs1, W1a, W1b)
    m1 = _sc_combine(y1, pe1, po1)
    ge2, go2, pos2, bexp2, _ = _router(m1, Wg2)
    pe2, po2 = pos2[:, 0], pos2[:, 1]
    xs2, gs2 = _sc_scatter(m1, ge2, go2, pe2, po2)
    y2 = _gmm(bexp2.reshape(-1), xs2, gs2, W2a, W2b)
    part = _sc_combine_reduce(y2, pe2, po2)
    out = _loss(y.astype(jnp.int32), xsum, part)
    return out[0, 0]


# speedup vs baseline: 1.1504x; 1.0154x over previous
import functools

import jax
import jax.numpy as jnp
from jax import lax
from jax.experimental import pallas as pl
from jax.experimental.pallas import tpu as pltpu
from jax.experimental.pallas import tpu_sc as plsc

_T = 2048
_D = 1024
_F = 1024
_E = 8
_M = 512
_CAP = 4096 + _E * _M
_NBLK = _CAP // _M


def _router_body(x_ref, wg_ref, ge_ref, go_ref, pos_ref, bexp_ref, xsum_ref):
    x = x_ref[...]
    wg = wg_ref[...]
    logits = lax.dot_general(
        x.astype(jnp.bfloat16), wg.astype(jnp.bfloat16),
        (((1,), (0,)), ((), ())),
        preferred_element_type=jnp.float32,
    )
    m = jnp.max(logits, axis=1, keepdims=True)
    p = jnp.exp(logits - m)
    p = p / jnp.sum(p, axis=1, keepdims=True)
    lane = lax.broadcasted_iota(jnp.int32, p.shape, 1)
    m1 = jnp.max(p, axis=1, keepdims=True)
    i1 = jnp.min(jnp.where(p == m1, lane, _E), axis=1, keepdims=True)
    p2 = jnp.where(lane == i1, -1.0, p)
    m2 = jnp.max(p2, axis=1, keepdims=True)
    i2 = jnp.min(jnp.where(p2 == m2, lane, _E), axis=1, keepdims=True)
    den = m1 + m2 + 1e-9
    g1 = m1 / den
    g2 = m2 / den
    ge_ref[...] = jnp.broadcast_to(g1, (_T, 128))
    go_ref[...] = jnp.broadcast_to(g2, (_T, 128))
    xsum_ref[...] = jnp.sum(x, axis=0, keepdims=True)

    onehot = (jnp.where(lane == i1, 1.0, 0.0)
              + jnp.where(lane == i2, 1.0, 0.0)).astype(jnp.bfloat16)
    ri = lax.broadcasted_iota(jnp.int32, (256, 256), 0)
    ci = lax.broadcasted_iota(jnp.int32, (256, 256), 1)
    ltri = (ci < ri).astype(jnp.bfloat16)
    parts = []
    carry = jnp.zeros((1, _E), jnp.float32)
    for j in range(_T // 256):
        oh = onehot[j * 256:(j + 1) * 256]
        r = lax.dot_general(
            ltri, oh, (((1,), (0,)), ((), ())),
            preferred_element_type=jnp.float32,
        ) + carry
        parts.append(r)
        carry = carry + jnp.sum(oh.astype(jnp.float32), axis=0, keepdims=True)
    ranks = jnp.concatenate(parts, axis=0)
    counts = carry
    pt = jnp.ceil(counts / _M) * _M
    r8 = lax.broadcasted_iota(jnp.int32, (_E, _E), 0)
    c8 = lax.broadcasted_iota(jnp.int32, (_E, _E), 1)
    utri = (r8 < c8).astype(jnp.float32)
    off = lax.dot_general(
        pt, utri, (((1,), (0,)), ((), ())),
        preferred_element_type=jnp.float32,
        precision=lax.Precision.HIGHEST,
    )
    off_b = jnp.broadcast_to(off, (_T, _E))
    rank1 = jnp.sum(jnp.where(lane == i1, ranks, 0.0), axis=1, keepdims=True)
    rank2 = jnp.sum(jnp.where(lane == i2, ranks, 0.0), axis=1, keepdims=True)
    off1 = jnp.sum(jnp.where(lane == i1, off_b, 0.0), axis=1, keepdims=True)
    off2 = jnp.sum(jnp.where(lane == i2, off_b, 0.0), axis=1, keepdims=True)
    pos1 = (off1 + rank1).astype(jnp.int32)
    pos2 = (off2 + rank2).astype(jnp.int32)
    pos_ref[...] = jnp.concatenate([pos1, pos2], axis=1)

    brow = lax.broadcasted_iota(jnp.int32, (_NBLK, _E), 0).astype(jnp.float32) * _M
    blane = lax.broadcasted_iota(jnp.int32, (_NBLK, _E), 1)
    off_nb = jnp.broadcast_to(off, (_NBLK, _E))
    pt_nb = jnp.broadcast_to(pt, (_NBLK, _E))
    member = jnp.logical_and(brow >= off_nb, brow < off_nb + pt_nb)
    bexp_ref[...] = jnp.sum(jnp.where(member, blane, 0), axis=1, keepdims=True)


def _router(xt, wg):
    return pl.pallas_call(
        _router_body,
        out_shape=(
            jax.ShapeDtypeStruct((_T, 128), jnp.float32),
            jax.ShapeDtypeStruct((_T, 128), jnp.float32),
            jax.ShapeDtypeStruct((_T, 2), jnp.int32),
            jax.ShapeDtypeStruct((_NBLK, 1), jnp.int32),
            jax.ShapeDtypeStruct((1, _D), jnp.float32),
        ),
    )(xt, wg)


def _gmm_body(bexp_ref, xs_ref, gs_ref, w1_ref, w2_ref, y_ref):
    xb = xs_ref[...].astype(jnp.bfloat16)
    w1 = w1_ref[0].astype(jnp.bfloat16)
    w2 = w2_ref[0].astype(jnp.bfloat16)
    h = lax.dot_general(
        xb, w1, (((1,), (0,)), ((), ())),
        preferred_element_type=jnp.float32,
    )
    g = gs_ref[...][:, 0:1]
    h = (jnp.maximum(h, 0.0) * g).astype(jnp.bfloat16)
    y_ref[...] = lax.dot_general(
        h, w2, (((1,), (0,)), ((), ())),
        preferred_element_type=jnp.float32,
    )


def _gmm(bexp, xs, gs, w1, w2):
    grid_spec = pltpu.PrefetchScalarGridSpec(
        num_scalar_prefetch=1,
        grid=(_NBLK,),
        in_specs=[
            pl.BlockSpec((_M, _D), lambda b, be: (b, 0)),
            pl.BlockSpec((_M, 128), lambda b, be: (b, 0)),
            pl.BlockSpec((1, _D, _F), lambda b, be: (be[b], 0, 0)),
            pl.BlockSpec((1, _F, _D), lambda b, be: (be[b], 0, 0)),
        ],
        out_specs=pl.BlockSpec((_M, _D), lambda b, be: (b, 0)),
    )
    return pl.pallas_call(
        _gmm_body,
        grid_spec=grid_spec,
        out_shape=jax.ShapeDtypeStruct((_CAP, _D), jnp.float32),
    )(bexp, xs, gs, w1, w2)


_NC, _NS = 2, 16
_NW = _NC * _NS
_TPW = _T // _NW


def _sc_scatter(x2d, ge, go, pos_e, pos_o):
    mesh = plsc.VectorSubcoreMesh(core_axis_name="c", subcore_axis_name="s")

    @functools.partial(
        pl.kernel, mesh=mesh,
        out_type=(
            jax.ShapeDtypeStruct((_CAP, _D), jnp.float32),
            jax.ShapeDtypeStruct((_CAP, 128), jnp.float32),
        ),
        scratch_types=[
            pltpu.VMEM((_TPW,), jnp.int32),
            pltpu.VMEM((_TPW,), jnp.int32),
            pltpu.VMEM((_TPW, _D), jnp.float32),
            pltpu.VMEM((_TPW, 128), jnp.float32),
            pltpu.VMEM((_TPW, 128), jnp.float32),
            pltpu.SemaphoreType.DMA,
            pltpu.SemaphoreType.DMA,
            pltpu.SemaphoreType.DMA,
            pltpu.SemaphoreType.DMA,
        ],
    )
    def k(x_hbm, ge_hbm, go_hbm, pe_hbm, po_hbm, xs_hbm, gs_hbm,
          pev, pov, xv, gev, gov, s1, s2, s3, s4):
        wid = lax.axis_index("s") * _NC + lax.axis_index("c")
        base = wid * _TPW
        pltpu.sync_copy(pe_hbm.at[pl.ds(base, _TPW)], pev)
        pltpu.sync_copy(po_hbm.at[pl.ds(base, _TPW)], pov)
        pltpu.sync_copy(x_hbm.at[pl.ds(base, _TPW)], xv)
        pltpu.sync_copy(ge_hbm.at[pl.ds(base, _TPW)], gev)
        pltpu.sync_copy(go_hbm.at[pl.ds(base, _TPW)], gov)
        c1 = pltpu.async_copy(xv, xs_hbm.at[pev], s1)
        c2 = pltpu.async_copy(xv, xs_hbm.at[pov], s2)
        c3 = pltpu.async_copy(gev, gs_hbm.at[pev], s3)
        c4 = pltpu.async_copy(gov, gs_hbm.at[pov], s4)
        c1.wait()
        c2.wait()
        c3.wait()
        c4.wait()

    return k(x2d, ge, go, pos_e, pos_o)


def _sc_combine(y, pos_e, pos_o):
    mesh = plsc.VectorSubcoreMesh(core_axis_name="c", subcore_axis_name="s")
    half = _TPW // 2

    @functools.partial(
        pl.kernel, mesh=mesh,
        out_type=jax.ShapeDtypeStruct((_T, _D), jnp.float32),
        scratch_types=[
            pltpu.VMEM((_TPW,), jnp.int32),
            pltpu.VMEM((_TPW,), jnp.int32),
            pltpu.VMEM((half, _D), jnp.float32),
            pltpu.VMEM((half, _D), jnp.float32),
            pltpu.SemaphoreType.DMA,
            pltpu.SemaphoreType.DMA,
        ],
    )
    def k(y_hbm, pe_hbm, po_hbm, out_hbm, pev, pov, rowsa, rowsb, sem1, sem2):
        wid = lax.axis_index("s") * _NC + lax.axis_index("c")
        base = wid * _TPW
        pltpu.sync_copy(pe_hbm.at[pl.ds(base, _TPW)], pev)
        pltpu.sync_copy(po_hbm.at[pl.ds(base, _TPW)], pov)
        for c in range(_TPW // half):
            ca = pltpu.async_copy(
                y_hbm.at[pev.at[pl.ds(half * c, half)]], rowsa, sem1)
            cb = pltpu.async_copy(
                y_hbm.at[pov.at[pl.ds(half * c, half)]], rowsb, sem2)
            ca.wait()
            cb.wait()

            def body(j, _):
                for l in range(_D // 16):
                    a = rowsa[j, pl.ds(16 * l, 16)]
                    b = rowsb[j, pl.ds(16 * l, 16)]
                    rowsa[j, pl.ds(16 * l, 16)] = a + b
                return 0

            lax.fori_loop(0, half, body, 0)
            pltpu.sync_copy(rowsa, out_hbm.at[pl.ds(base + half * c, half)])

    return k(y, pos_e, pos_o)


def _sc_combine_reduce(y, pos_e, pos_o):
    mesh = plsc.VectorSubcoreMesh(core_axis_name="c", subcore_axis_name="s")
    half = _TPW // 2

    @functools.partial(
        pl.kernel, mesh=mesh,
        out_type=jax.ShapeDtypeStruct((_NW, _D), jnp.float32),
        scratch_types=[
            pltpu.VMEM((_TPW,), jnp.int32),
            pltpu.VMEM((_TPW,), jnp.int32),
            pltpu.VMEM((half, _D), jnp.float32),
            pltpu.VMEM((half, _D), jnp.float32),
            pltpu.VMEM((1, _D), jnp.float32),
            pltpu.SemaphoreType.DMA,
            pltpu.SemaphoreType.DMA,
        ],
    )
    def k(y_hbm, pe_hbm, po_hbm, out_hbm, pev, pov, rowsa, rowsb, acc,
          sem1, sem2):
        wid = lax.axis_index("s") * _NC + lax.axis_index("c")
        base = wid * _TPW
        pltpu.sync_copy(pe_hbm.at[pl.ds(base, _TPW)], pev)
        pltpu.sync_copy(po_hbm.at[pl.ds(base, _TPW)], pov)
        zero = jnp.zeros((16,), jnp.float32)
        for l in range(_D // 16):
            acc[0, pl.ds(16 * l, 16)] = zero
        for c in range(_TPW // half):
            ca = pltpu.async_copy(
                y_hbm.at[pev.at[pl.ds(half * c, half)]], rowsa, sem1)
            cb = pltpu.async_copy(
                y_hbm.at[pov.at[pl.ds(half * c, half)]], rowsb, sem2)
            ca.wait()
            cb.wait()

            def body(j, _):
                for l in range(_D // 16):
                    a = rowsa[j, pl.ds(16 * l, 16)]
                    b = rowsb[j, pl.ds(16 * l, 16)]
                    rowsa[j, pl.ds(16 * l, 16)] = a + b
                return 0

            lax.fori_loop(0, half, body, 0)
            for s in (16, 8, 4, 2, 1):

                def tbody(j, _, s=s):
                    for l in range(_D // 16):
                        rowsa[j, pl.ds(16 * l, 16)] = (
                            rowsa[j, pl.ds(16 * l, 16)]
                            + rowsa[j + s, pl.ds(16 * l, 16)])
                    return 0

                lax.fori_loop(0, s, tbody, 0)
            if True:
                for l in range(_D // 16):
                    acc[0, pl.ds(16 * l, 16)] = (
                        acc[0, pl.ds(16 * l, 16)] + rowsa[0, pl.ds(16 * l, 16)])
        pltpu.sync_copy(acc, out_hbm.at[pl.ds(wid, 1)])

    return k(y, pos_e, pos_o)


def _loss_body(y_ref, xsum_ref, part_ref, out_ref):
    sent = (xsum_ref[...] + jnp.sum(part_ref[...], axis=0, keepdims=True))
    sent = sent / float(_T)
    mx = jnp.max(sent)
    lse = jnp.log(jnp.sum(jnp.exp(sent - mx))) + mx
    yv = y_ref[0]
    lane = lax.broadcasted_iota(jnp.int32, sent.shape, 1)
    picked = jnp.sum(jnp.where(lane == yv, sent, 0.0))
    out_ref[0, 0] = lse - picked


def _loss(y, xsum, partials):
    return pl.pallas_call(
        _loss_body,
        in_specs=[
            pl.BlockSpec(memory_space=pltpu.SMEM),
            pl.BlockSpec((1, _D), lambda: (0, 0)),
            pl.BlockSpec((_NW, _D), lambda: (0, 0)),
        ],
        out_specs=pl.BlockSpec(memory_space=pltpu.SMEM),
        out_shape=jax.ShapeDtypeStruct((1, 1), jnp.float32),
    )(y, xsum, partials)


def kernel(x, y, Wg1, W1a, W1b, Wg2, W2a, W2b):
    xt = x.reshape(_T, _D)
    ge1, go1, pos1, bexp1, xsum = _router(xt, Wg1)
    pe1, po1 = pos1[:, 0], pos1[:, 1]
    xs1, gs1 = _sc_scatter(xt, ge1, go1, pe1, po1)
    y1 = _gmm(bexp1.reshape(-1), xs1, gs1, W1a, W1b)
    m1 = _sc_combine(y1, pe1, po1)
    ge2, go2, pos2, bexp2, _ = _router(m1, Wg2)
    pe2, po2 = pos2[:, 0], pos2[:, 1]
    xs2, gs2 = _sc_scatter(m1, ge2, go2, pe2, po2)
    y2 = _gmm(bexp2.reshape(-1), xs2, gs2, W2a, W2b)
    part = _sc_combine_reduce(y2, pe2, po2)
    out = _loss(y.astype(jnp.int32), xsum, part)
    return out[0, 0]
